# Initial kernel scaffold; baseline (speedup 1.0000x reference)
import jax
import jax.numpy as jnp
from jax.experimental import pallas as pl


def kernel(x, edge_index, W_l1, W_r1, b1, W_l2, W_r2, b2, W_l3, W_r3, b3):
    raise NotImplementedError("write your pallas kernel here")


# speedup vs baseline: 5.3184x; 5.3184x over previous
import functools

import jax
import jax.numpy as jnp
from jax import lax
from jax.experimental import pallas as pl
from jax.experimental.pallas import tpu as pltpu
from jax.experimental.pallas import tpu_sc as plsc

N_NODES = 10000
N_EDGES = 320000
D_IN = 128

NC, NS = 2, 16
NW = NC * NS
CHUNK = 128
CHUNKS_PER_W = -(-N_EDGES // (NW * CHUNK))
E_PAD = NW * CHUNKS_PER_W * CHUNK
N_PAD = 10240
ROWS_PER_TILE = N_PAD // NS


def _sc_aggregate(width, with_deg):
  mesh = plsc.VectorSubcoreMesh(core_axis_name="c", subcore_axis_name="s")
  out_type = [jax.ShapeDtypeStruct((NC, N_PAD, width), jnp.float32)]
  if with_deg:
    out_type.append(jax.ShapeDtypeStruct((NC, N_PAD), jnp.float32))
  scratch = [
      pltpu.VMEM((CHUNK,), jnp.int32),
      pltpu.VMEM((CHUNK,), jnp.int32),
      pltpu.VMEM((CHUNK, width), jnp.float32),
      pltpu.VMEM((CHUNK, width), jnp.float32),
      pltpu.VMEM_SHARED((N_PAD, width), jnp.float32),
      pltpu.SemaphoreType.DMA,
  ]
  if with_deg:
    scratch += [
        pltpu.VMEM((CHUNK,), jnp.float32),
        pltpu.VMEM((ROWS_PER_TILE,), jnp.float32),
        pltpu.VMEM_SHARED((N_PAD,), jnp.float32),
    ]

  def body(table, src, dst, *rest):
    if with_deg:
      (agg_out, deg_out, src_idx, dst_idx, rows, zbuf, acc, sem,
       ones, zdeg, dacc) = rest
    else:
      (agg_out, src_idx, dst_idx, rows, zbuf, acc, sem) = rest
    c = lax.axis_index("c")
    s = lax.axis_index("s")
    wid = s * NC + c
    tile_base = s * ROWS_PER_TILE

    zero16 = jnp.zeros((16,), jnp.float32)

    def zrow(i, carry):
      for j in range(width // 16):
        zbuf[i, pl.ds(j * 16, 16)] = zero16
      return carry

    lax.fori_loop(0, CHUNK, zrow, 0)
    for k in range(ROWS_PER_TILE // CHUNK):
      pltpu.sync_copy(zbuf, acc.at[pl.ds(tile_base + k * CHUNK, CHUNK)])
    if with_deg:
      def zdrow(i, carry):
        zdeg[pl.ds(i * 16, 16)] = zero16
        return carry

      lax.fori_loop(0, ROWS_PER_TILE // 16, zdrow, 0)
      one16 = jnp.ones((16,), jnp.float32)
      for j in range(CHUNK // 16):
        ones[pl.ds(j * 16, 16)] = one16
      pltpu.sync_copy(zdeg, dacc.at[pl.ds(tile_base, ROWS_PER_TILE)])
    plsc.subcore_barrier()

    base = wid * CHUNKS_PER_W * CHUNK

    def step(j, carry):
      e0 = base + j * CHUNK
      pltpu.sync_copy(src.at[pl.ds(e0, CHUNK)], src_idx)
      pltpu.sync_copy(dst.at[pl.ds(e0, CHUNK)], dst_idx)
      pltpu.async_copy(table.at[src_idx], rows, sem).wait()
      pltpu.sync_copy(rows, acc.at[dst_idx], add=True)
      if with_deg:
        pltpu.sync_copy(ones, dacc.at[dst_idx], add=True)
      return carry

    lax.fori_loop(0, CHUNKS_PER_W, step, 0)
    plsc.subcore_barrier()

    pltpu.sync_copy(acc.at[pl.ds(tile_base, ROWS_PER_TILE)],
                    agg_out.at[c, pl.ds(tile_base, ROWS_PER_TILE)])
    if with_deg:
      pltpu.sync_copy(dacc.at[pl.ds(tile_base, ROWS_PER_TILE)],
                      deg_out.at[c, pl.ds(tile_base, ROWS_PER_TILE)])

  return pl.kernel(
      body,
      out_type=tuple(out_type) if with_deg else out_type[0],
      mesh=mesh,
      scratch_types=scratch,
      compiler_params=pltpu.CompilerParams(use_tc_tiling_on_sc=(width % 128 == 0)),
  )


_sc_agg_deg = _sc_aggregate(128, True)
_sc_agg_128 = _sc_aggregate(128, False)
_sc_agg_48 = _sc_aggregate(48, False)


def _leaky(t):
  return jnp.where(t > 0, t, 0.01 * t)


BLK = 1024
GRID = N_PAD // BLK


def _full(shape):
  return pl.BlockSpec(shape, lambda i: (0,) * len(shape))


def _rows2(w):
  return pl.BlockSpec((BLK, w), lambda i: (i, 0))


def _agg_spec(w):
  return pl.BlockSpec((NC, BLK, w), lambda i: (0, i, 0))


_DEG_SPEC = pl.BlockSpec((NC, BLK), lambda i: (0, i))


def _tc1_body(agg_ref, deg_ref, x_ref, wl1, wr1, b1r, wl2, wr2, b2r,
              y2_ref, s2_ref):
  d = jnp.maximum(deg_ref[0, :] + deg_ref[1, :], 1.0)
  mean = (agg_ref[0] + agg_ref[1]) / d[:, None]
  t = jnp.dot(mean, wl1[...], preferred_element_type=jnp.float32)
  t = t + jnp.dot(x_ref[...], wr1[...], preferred_element_type=jnp.float32)
  h1 = _leaky(t + b1r[...])
  y2_ref[...] = jnp.dot(h1, wl2[...], preferred_element_type=jnp.float32)
  s2_ref[...] = (jnp.dot(h1, wr2[...], preferred_element_type=jnp.float32)
                 + b2r[...])


def _tc2_body(agg_ref, deg_ref, s2_ref, wl3, wr3, b3r, y3_ref, s3_ref):
  d = jnp.maximum(deg_ref[0, :] + deg_ref[1, :], 1.0)
  h2 = _leaky((agg_ref[0] + agg_ref[1]) / d[:, None] + s2_ref[...])
  y3_ref[...] = jnp.dot(h2, wl3[...], preferred_element_type=jnp.float32)
  s3_ref[...] = (jnp.dot(h2, wr3[...], preferred_element_type=jnp.float32)
                 + b3r[...])


def _tc3_body(agg_ref, deg_ref, s3_ref, out_ref):
  d = jnp.maximum(deg_ref[0, :] + deg_ref[1, :], 1.0)
  z = (agg_ref[0] + agg_ref[1])[:, :40] / d[:, None] + s3_ref[...]
  m = jnp.max(z, axis=1, keepdims=True)
  e = jnp.exp(z - m)
  lse = jnp.log(jnp.sum(e, axis=1, keepdims=True))
  out_ref[...] = z - m - lse


_tc1 = pl.pallas_call(
    _tc1_body,
    grid=(GRID,),
    in_specs=[_agg_spec(128), _DEG_SPEC, _rows2(128), _full((128, 256)),
              _full((128, 256)), _full((1, 256)), _full((256, 128)),
              _full((256, 128)), _full((1, 128))],
    out_specs=[_rows2(128), _rows2(128)],
    out_shape=[jax.ShapeDtypeStruct((N_PAD, 128), jnp.float32),
               jax.ShapeDtypeStruct((N_PAD, 128), jnp.float32)],
)

_tc2 = pl.pallas_call(
    _tc2_body,
    grid=(GRID,),
    in_specs=[_agg_spec(128), _DEG_SPEC, _rows2(128), _full((128, 48)),
              _full((128, 40)), _full((1, 40))],
    out_specs=[_rows2(48), _rows2(40)],
    out_shape=[jax.ShapeDtypeStruct((N_PAD, 48), jnp.float32),
               jax.ShapeDtypeStruct((N_PAD, 40), jnp.float32)],
)

_tc3 = pl.pallas_call(
    _tc3_body,
    grid=(GRID,),
    in_specs=[_agg_spec(48), _DEG_SPEC, _rows2(40)],
    out_specs=_rows2(40),
    out_shape=jax.ShapeDtypeStruct((N_PAD, 40), jnp.float32),
)


@jax.jit
def kernel(x, edge_index, W_l1, W_r1, b1, W_l2, W_r2, b2, W_l3, W_r3, b3):
  src = edge_index[0].astype(jnp.int32)
  dst = edge_index[1].astype(jnp.int32)
  pad_e = E_PAD - N_EDGES
  src_p = jnp.concatenate([src, jnp.zeros((pad_e,), jnp.int32)])
  dst_p = jnp.concatenate([dst, jnp.full((pad_e,), N_NODES, jnp.int32)])
  x_p = jnp.zeros((N_PAD, D_IN), jnp.float32).at[:N_NODES].set(x)

  agg1, deg = _sc_agg_deg(x_p, src_p, dst_p)
  y2, s2 = _tc1(agg1, deg, x_p, W_l1, W_r1, b1.reshape(1, -1),
                W_l2, W_r2, b2.reshape(1, -1))
  agg2 = _sc_agg_128(y2, src_p, dst_p)
  wl3p = jnp.pad(W_l3, ((0, 0), (0, 8)))
  y3, s3 = _tc2(agg2, deg, s2, wl3p, W_r3, b3.reshape(1, -1))
  agg3 = _sc_agg_48(y3, src_p, dst_p)
  out = _tc3(agg3, deg, s3)
  return out[:N_NODES]


# speedup vs baseline: 5.3533x; 1.0066x over previous
import functools

import jax
import jax.numpy as jnp
from jax import lax
from jax.experimental import pallas as pl
from jax.experimental.pallas import tpu as pltpu
from jax.experimental.pallas import tpu_sc as plsc

N_NODES = 10000
N_EDGES = 320000
D_IN = 128

NC, NS = 2, 16
NW = NC * NS
CHUNK = 128
CHUNKS_PER_W = 80
E_PAD = NW * CHUNKS_PER_W * CHUNK
N_PAD = 10240
ROWS_PER_TILE = N_PAD // NS


def _sc_aggregate(width, with_deg):
  mesh = plsc.VectorSubcoreMesh(core_axis_name="c", subcore_axis_name="s")
  out_type = [jax.ShapeDtypeStruct((NC, N_PAD, width), jnp.float32)]
  if with_deg:
    out_type.append(jax.ShapeDtypeStruct((NC, N_PAD), jnp.float32))
  scratch = [
      pltpu.VMEM((CHUNK,), jnp.int32),
      pltpu.VMEM((CHUNK,), jnp.int32),
      pltpu.VMEM((CHUNK, width), jnp.float32),
      pltpu.VMEM((CHUNK,), jnp.int32),
      pltpu.VMEM((CHUNK,), jnp.int32),
      pltpu.VMEM((CHUNK, width), jnp.float32),
      pltpu.VMEM_SHARED((N_PAD, width), jnp.float32),
      pltpu.SemaphoreType.DMA,
      pltpu.SemaphoreType.DMA,
  ]
  if with_deg:
    scratch += [
        pltpu.VMEM((CHUNK,), jnp.float32),
        pltpu.VMEM((ROWS_PER_TILE,), jnp.float32),
        pltpu.VMEM_SHARED((N_PAD,), jnp.float32),
    ]

  def body(table, src, dst, *rest):
    if with_deg:
      (agg_out, deg_out, srci0, dsti0, rows0, srci1, dsti1, rows1,
       acc, gsem0, gsem1, ones, zdeg, dacc) = rest
    else:
      (agg_out, srci0, dsti0, rows0, srci1, dsti1, rows1,
       acc, gsem0, gsem1) = rest
    zbuf = rows0
    bufs = ((srci0, dsti0, rows0, gsem0), (srci1, dsti1, rows1, gsem1))
    c = lax.axis_index("c")
    s = lax.axis_index("s")
    wid = s * NC + c
    tile_base = s * ROWS_PER_TILE

    zero16 = jnp.zeros((16,), jnp.float32)

    def zrow(i, carry):
      for j in range(width // 16):
        zbuf[i, pl.ds(j * 16, 16)] = zero16
      return carry

    lax.fori_loop(0, CHUNK, zrow, 0)
    for k in range(ROWS_PER_TILE // CHUNK):
      pltpu.sync_copy(zbuf, acc.at[pl.ds(tile_base + k * CHUNK, CHUNK)])
    if with_deg:
      def zdrow(i, carry):
        zdeg[pl.ds(i * 16, 16)] = zero16
        return carry

      lax.fori_loop(0, ROWS_PER_TILE // 16, zdrow, 0)
      one16 = jnp.ones((16,), jnp.float32)
      for j in range(CHUNK // 16):
        ones[pl.ds(j * 16, 16)] = one16
      pltpu.sync_copy(zdeg, dacc.at[pl.ds(tile_base, ROWS_PER_TILE)])
    plsc.subcore_barrier()

    base = wid * CHUNKS_PER_W * CHUNK

    for b in range(2):
      srci, dsti, rows, gsem = bufs[b]
      pltpu.sync_copy(src.at[pl.ds(base + b * CHUNK, CHUNK)], srci)
      pltpu.sync_copy(dst.at[pl.ds(base + b * CHUNK, CHUNK)], dsti)
      pltpu.async_copy(table.at[srci], rows, gsem)

    def step(i, carry):
      jj = i * 2
      for b in range(2):
        j = jj + b
        srci, dsti, rows, gsem = bufs[b]
        pltpu.make_async_copy(table.at[srci], rows, gsem).wait()
        pltpu.sync_copy(rows, acc.at[dsti], add=True)
        if with_deg:
          pltpu.sync_copy(ones, dacc.at[dsti], add=True)

        @pl.when(j + 2 < CHUNKS_PER_W)
        def _():
          e0 = base + (j + 2) * CHUNK
          pltpu.sync_copy(src.at[pl.ds(e0, CHUNK)], srci)
          pltpu.sync_copy(dst.at[pl.ds(e0, CHUNK)], dsti)
          pltpu.async_copy(table.at[srci], rows, gsem)

      return carry

    lax.fori_loop(0, CHUNKS_PER_W // 2, step, 0)
    plsc.subcore_barrier()

    pltpu.sync_copy(acc.at[pl.ds(tile_base, ROWS_PER_TILE)],
                    agg_out.at[c, pl.ds(tile_base, ROWS_PER_TILE)])
    if with_deg:
      pltpu.sync_copy(dacc.at[pl.ds(tile_base, ROWS_PER_TILE)],
                      deg_out.at[c, pl.ds(tile_base, ROWS_PER_TILE)])

  return pl.kernel(
      body,
      out_type=tuple(out_type) if with_deg else out_type[0],
      mesh=mesh,
      scratch_types=scratch,
      compiler_params=pltpu.CompilerParams(use_tc_tiling_on_sc=(width % 128 == 0)),
  )


_sc_agg_deg = _sc_aggregate(128, True)
_sc_agg_128 = _sc_aggregate(128, False)
_sc_agg_48 = _sc_aggregate(48, False)


def _leaky(t):
  return jnp.where(t > 0, t, 0.01 * t)


BLK = 1024
GRID = N_PAD // BLK


def _full(shape):
  return pl.BlockSpec(shape, lambda i: (0,) * len(shape))


def _rows2(w):
  return pl.BlockSpec((BLK, w), lambda i: (i, 0))


def _agg_spec(w):
  return pl.BlockSpec((NC, BLK, w), lambda i: (0, i, 0))


_DEG_SPEC = pl.BlockSpec((NC, BLK), lambda i: (0, i))


def _tc1_body(agg_ref, deg_ref, x_ref, wl1, wr1, b1r, wl2, wr2, b2r,
              y2_ref, s2_ref):
  d = jnp.maximum(deg_ref[0, :] + deg_ref[1, :], 1.0)
  mean = (agg_ref[0] + agg_ref[1]) / d[:, None]
  t = jnp.dot(mean, wl1[...], preferred_element_type=jnp.float32)
  t = t + jnp.dot(x_ref[...], wr1[...], preferred_element_type=jnp.float32)
  h1 = _leaky(t + b1r[...])
  y2_ref[...] = jnp.dot(h1, wl2[...], preferred_element_type=jnp.float32)
  s2_ref[...] = (jnp.dot(h1, wr2[...], preferred_element_type=jnp.float32)
                 + b2r[...])


def _tc2_body(agg_ref, deg_ref, s2_ref, wl3, wr3, b3r, y3_ref, s3_ref):
  d = jnp.maximum(deg_ref[0, :] + deg_ref[1, :], 1.0)
  h2 = _leaky((agg_ref[0] + agg_ref[1]) / d[:, None] + s2_ref[...])
  y3_ref[...] = jnp.dot(h2, wl3[...], preferred_element_type=jnp.float32)
  s3_ref[...] = (jnp.dot(h2, wr3[...], preferred_element_type=jnp.float32)
                 + b3r[...])


def _tc3_body(agg_ref, deg_ref, s3_ref, out_ref):
  d = jnp.maximum(deg_ref[0, :] + deg_ref[1, :], 1.0)
  z = (agg_ref[0] + agg_ref[1])[:, :40] / d[:, None] + s3_ref[...]
  m = jnp.max(z, axis=1, keepdims=True)
  e = jnp.exp(z - m)
  lse = jnp.log(jnp.sum(e, axis=1, keepdims=True))
  out_ref[...] = z - m - lse


_tc1 = pl.pallas_call(
    _tc1_body,
    grid=(GRID,),
    in_specs=[_agg_spec(128), _DEG_SPEC, _rows2(128), _full((128, 256)),
              _full((128, 256)), _full((1, 256)), _full((256, 128)),
              _full((256, 128)), _full((1, 128))],
    out_specs=[_rows2(128), _rows2(128)],
    out_shape=[jax.ShapeDtypeStruct((N_PAD, 128), jnp.float32),
               jax.ShapeDtypeStruct((N_PAD, 128), jnp.float32)],
)

_tc2 = pl.pallas_call(
    _tc2_body,
    grid=(GRID,),
    in_specs=[_agg_spec(128), _DEG_SPEC, _rows2(128), _full((128, 48)),
              _full((128, 40)), _full((1, 40))],
    out_specs=[_rows2(48), _rows2(40)],
    out_shape=[jax.ShapeDtypeStruct((N_PAD, 48), jnp.float32),
               jax.ShapeDtypeStruct((N_PAD, 40), jnp.float32)],
)

_tc3 = pl.pallas_call(
    _tc3_body,
    grid=(GRID,),
    in_specs=[_agg_spec(48), _DEG_SPEC, _rows2(40)],
    out_specs=_rows2(40),
    out_shape=jax.ShapeDtypeStruct((N_PAD, 40), jnp.float32),
)


@jax.jit
def kernel(x, edge_index, W_l1, W_r1, b1, W_l2, W_r2, b2, W_l3, W_r3, b3):
  src = edge_index[0].astype(jnp.int32)
  dst = edge_index[1].astype(jnp.int32)
  pad_e = E_PAD - N_EDGES
  src_p = jnp.concatenate([src, jnp.zeros((pad_e,), jnp.int32)])
  dst_p = jnp.concatenate([dst, jnp.full((pad_e,), N_NODES, jnp.int32)])
  x_p = jnp.zeros((N_PAD, D_IN), jnp.float32).at[:N_NODES].set(x)

  agg1, deg = _sc_agg_deg(x_p, src_p, dst_p)
  y2, s2 = _tc1(agg1, deg, x_p, W_l1, W_r1, b1.reshape(1, -1),
                W_l2, W_r2, b2.reshape(1, -1))
  agg2 = _sc_agg_128(y2, src_p, dst_p)
  wl3p = jnp.pad(W_l3, ((0, 0), (0, 8)))
  y3, s3 = _tc2(agg2, deg, s2, wl3p, W_r3, b3.reshape(1, -1))
  agg3 = _sc_agg_48(y3, src_p, dst_p)
  out = _tc3(agg3, deg, s3)
  return out[:N_NODES]
